# baseline (speedup 1.0000x reference)
import functools

import jax
import jax.numpy as jnp
from jax import lax
from jax.experimental import pallas as pl
from jax.experimental.pallas import tpu as pltpu
from jax.experimental.pallas import tpu_sc as plsc

N = 10000
E = 160000
D = 128
H = 512
NUM_E = 3

TE = 512
EP = 163840
NB = EP // TE

NC, NS = 2, 16
NW = NC * NS
CHUNK = 128


NBUF = 5


CH = 64


def _sc_route_sum_rows(table, src_idx, dst_idx, pos2d, out_rows):
    per_w = src_idx.shape[0] // NW
    n_chunks = per_w // CH
    assert per_w % CH == 0 and n_chunks % NBUF == 0 and n_chunks % 8 == 0
    n_rounds = n_chunks // NBUF
    mesh = plsc.VectorSubcoreMesh(
        core_axis_name="c", subcore_axis_name="s",
        num_cores=NC, num_subcores=NS)

    @functools.partial(
        pl.kernel,
        out_type=jax.ShapeDtypeStruct((out_rows, D), jnp.float32),
        mesh=mesh,
        scratch_types=[
            pltpu.VMEM((per_w,), jnp.int32),
            pltpu.VMEM((per_w,), jnp.int32),
            pltpu.VMEM((n_chunks, CH), jnp.int32),
            pltpu.VMEM((NBUF, CH, D), jnp.float32),
            pltpu.VMEM((NBUF, CH, D), jnp.float32),
            pltpu.SemaphoreType.DMA((NBUF,)),
            pltpu.SemaphoreType.DMA((NBUF,)),
            pltpu.SemaphoreType.DMA((NBUF,)),
        ],
    )
    def route_kernel(table_hbm, src_hbm, dst_hbm, pos_hbm, out_hbm,
                     src_v, dst_v, pos_v, rows_s, rows_d, gsems, gsemd, ssem):
        wid = lax.axis_index("s") * NC + lax.axis_index("c")
        base0 = wid * per_w
        pltpu.sync_copy(src_hbm.at[pl.ds(base0, per_w)], src_v)
        pltpu.sync_copy(dst_hbm.at[pl.ds(base0, per_w)], dst_v)
        pltpu.sync_copy(pos_hbm.at[pl.ds(wid * n_chunks, n_chunks), :], pos_v)

        def issue_gathers(c, b):
            pltpu.async_copy(
                table_hbm.at[src_v.at[pl.ds(c * CH, CH)]],
                rows_s.at[b], gsems.at[b])
            pltpu.async_copy(
                table_hbm.at[dst_v.at[pl.ds(c * CH, CH)]],
                rows_d.at[b], gsemd.at[b])

        def wait_gathers(b):
            pltpu.make_async_copy(
                table_hbm.at[src_v.at[pl.ds(0, CH)]],
                rows_s.at[b], gsems.at[b]).wait()
            pltpu.make_async_copy(
                table_hbm.at[dst_v.at[pl.ds(0, CH)]],
                rows_d.at[b], gsemd.at[b]).wait()

        def add_rows(b):
            bs = rows_s.at[b]
            bd = rows_d.at[b]

            def row_body(r, carry):
                for k in range(D // 16):
                    bs[r, pl.ds(k * 16, 16)] = (
                        bs[r, pl.ds(k * 16, 16)] + bd[r, pl.ds(k * 16, 16)])
                return carry

            lax.fori_loop(0, CH, row_body, 0)

        def issue_scatter(c, b):
            pltpu.async_copy(
                rows_s.at[b], out_hbm.at[pos_v.at[c]], ssem.at[b])

        def wait_scatter(b):
            pltpu.make_async_copy(
                rows_s.at[b], out_hbm.at[pos_v.at[0]], ssem.at[b]).wait()

        for b in range(NBUF):
            issue_gathers(b, b)

        def round_body(o, carry):
            c0 = o * NBUF
            for b in range(NBUF):
                wait_gathers(b)
                add_rows(b)
                issue_scatter(c0 + b, b)
            for b in range(NBUF):
                wait_scatter(b)
                issue_gathers(c0 + NBUF + b, b)
            return carry

        lax.fori_loop(0, n_rounds - 1, round_body, 0)

        c0 = (n_rounds - 1) * NBUF
        for b in range(NBUF):
            wait_gathers(b)
            add_rows(b)
            issue_scatter(c0 + b, b)
        for b in range(NBUF):
            wait_scatter(b)

    return route_kernel(table, src_idx, dst_idx, pos2d)


def _sc_gather_rows(table, idx, rows_total, clamp_max=0):
    per_w = rows_total // NW
    n_chunks = per_w // CHUNK
    assert per_w % CHUNK == 0 and n_chunks % NBUF == 0
    n_rounds = n_chunks // NBUF
    mesh = plsc.VectorSubcoreMesh(
        core_axis_name="c", subcore_axis_name="s",
        num_cores=NC, num_subcores=NS)

    @functools.partial(
        pl.kernel,
        out_type=jax.ShapeDtypeStruct((rows_total, D), jnp.float32),
        mesh=mesh,
        scratch_types=[
            pltpu.VMEM((per_w,), jnp.int32),
            pltpu.VMEM((NBUF, CHUNK, D), jnp.float32),
            pltpu.SemaphoreType.DMA((NBUF,)),
            pltpu.SemaphoreType.DMA((NBUF,)),
        ],
    )
    def gather_kernel(table_hbm, idx_hbm, out_hbm, idx_v, rows_v, gsem, ssem):
        wid = lax.axis_index("s") * NC + lax.axis_index("c")
        base0 = wid * per_w
        pltpu.sync_copy(idx_hbm.at[pl.ds(base0, per_w)], idx_v)

        if clamp_max > 0:
            def clamp_body(i, carry):
                v = idx_v[pl.ds(i * 16, 16)]
                idx_v[pl.ds(i * 16, 16)] = jnp.minimum(
                    jnp.maximum(v, 0), clamp_max - 1)
                return carry

            lax.fori_loop(0, per_w // 16, clamp_body, 0)

        def issue_gather(c, b):
            pltpu.async_copy(
                table_hbm.at[idx_v.at[pl.ds(c * CHUNK, CHUNK)]],
                rows_v.at[b], gsem.at[b])

        def wait_gather(b):
            pltpu.make_async_copy(
                table_hbm.at[idx_v.at[pl.ds(0, CHUNK)]],
                rows_v.at[b], gsem.at[b]).wait()

        def issue_store(c, b):
            pltpu.async_copy(
                rows_v.at[b],
                out_hbm.at[pl.ds(base0 + c * CHUNK, CHUNK), :], ssem.at[b])

        def wait_store(b):
            pltpu.make_async_copy(
                rows_v.at[b],
                out_hbm.at[pl.ds(base0, CHUNK), :], ssem.at[b]).wait()

        for b in range(NBUF):
            issue_gather(b, b)

        def round_body(o, carry):
            c0 = o * NBUF
            for b in range(NBUF):
                wait_gather(b)
                issue_store(c0 + b, b)
            for b in range(NBUF):
                wait_store(b)
                issue_gather(c0 + NBUF + b, b)
            return carry

        lax.fori_loop(0, n_rounds - 1, round_body, 0)

        c0 = (n_rounds - 1) * NBUF
        for b in range(NBUF):
            wait_gather(b)
            issue_store(c0 + b, b)
        for b in range(NBUF):
            wait_store(b)

    return gather_kernel(table, idx)


def _mlp_body(be_ref, hs_ref, w1_ref, b1_ref, w2_ref, b2_ref, o_ref):
    h = hs_ref[...].astype(jnp.bfloat16)
    z = jnp.dot(h, w1_ref[0], preferred_element_type=jnp.float32)
    z = jnp.maximum(z + b1_ref[0], 0.0).astype(jnp.bfloat16)
    o_ref[...] = jnp.dot(z, w2_ref[0], preferred_element_type=jnp.float32) + b2_ref[0]


def _routed_mlp(block_expert, g, W1, b1, W2, b2):
    grid_spec = pltpu.PrefetchScalarGridSpec(
        num_scalar_prefetch=1,
        grid=(NB,),
        in_specs=[
            pl.BlockSpec((TE, D), lambda i, be: (i, 0)),
            pl.BlockSpec((1, D, H), lambda i, be: (be[i], 0, 0)),
            pl.BlockSpec((1, 1, H), lambda i, be: (be[i], 0, 0)),
            pl.BlockSpec((1, H, D), lambda i, be: (be[i], 0, 0)),
            pl.BlockSpec((1, 1, D), lambda i, be: (be[i], 0, 0)),
        ],
        out_specs=pl.BlockSpec((TE, D), lambda i, be: (i, 0)),
    )
    return pl.pallas_call(
        _mlp_body,
        grid_spec=grid_spec,
        out_shape=jax.ShapeDtypeStruct((EP, D), jnp.float32),
    )(block_expert, g, W1.astype(jnp.bfloat16), b1.reshape(NUM_E, 1, H),
      W2.astype(jnp.bfloat16), b2.reshape(NUM_E, 1, D))


def kernel(x, edge_index, edge_vec, W1, b1, W2, b2):
    src = edge_index[0]
    dst = edge_index[1]
    lengths = jnp.sqrt(jnp.sum(edge_vec * edge_vec, axis=-1))
    eid = (lengths >= 1.3).astype(jnp.int32) + (lengths >= 2.0).astype(jnp.int32)

    onehot = (eid[:, None] == jnp.arange(NUM_E, dtype=jnp.int32)[None, :])
    csum = jnp.cumsum(onehot.astype(jnp.int32), axis=0)
    counts = csum[-1]
    rank = jnp.take_along_axis(csum, eid[:, None], axis=1)[:, 0] - 1
    nb_g = (counts + TE - 1) // TE
    off = jnp.concatenate(
        [jnp.zeros((1,), jnp.int32), jnp.cumsum(nb_g[:2] * TE).astype(jnp.int32)])
    padded_pos = off[eid] + rank

    pad_n = NW * (-(-E // (NW * CH * 8 * NBUF)) * CH * 8 * NBUF) - E
    pos_cat = jnp.concatenate(
        [padded_pos, EP + jnp.arange(pad_n, dtype=jnp.int32)]).reshape(-1, CH)
    zpad = jnp.zeros((pad_n,), jnp.int32)
    hsum = _sc_route_sum_rows(
        x, jnp.concatenate([src, zpad]), jnp.concatenate([dst, zpad]),
        pos_cat, EP + pad_n)

    blk = jnp.arange(NB, dtype=jnp.int32) * TE
    block_expert = (blk >= off[1]).astype(jnp.int32) + (blk >= off[2]).astype(jnp.int32)

    out_padded = _routed_mlp(block_expert, hsum, W1, b1, W2, b2)

    gpos = jnp.concatenate([padded_pos, jnp.zeros((EP - E,), jnp.int32)])
    res_pad = _sc_gather_rows(out_padded, gpos, EP)
    return res_pad[:E]


# speedup vs baseline: 1.5061x; 1.5061x over previous
import functools

import jax
import jax.numpy as jnp
from jax import lax
from jax.experimental import pallas as pl
from jax.experimental.pallas import tpu as pltpu
from jax.experimental.pallas import tpu_sc as plsc

N = 10000
E = 160000
D = 128
H = 512
NUM_E = 3

TE = 512
EP = 163840
NB = EP // TE

NC, NS = 2, 16
NW = NC * NS
CHUNK = 128


NBUF = 5


CH = 32


def _sc_route_sum_rows(table, src_idx, dst_idx, pos2d, out_rows):
    per_w = src_idx.shape[0] // NW
    n_chunks = per_w // CH
    assert per_w % CH == 0 and n_chunks % 8 == 0
    NBR = 2
    n_rounds = n_chunks // NBR
    n_tab = table.shape[0]
    assert n_tab % (NS * 8) == 0
    tab_per_tile = n_tab // NS
    mesh = plsc.VectorSubcoreMesh(
        core_axis_name="c", subcore_axis_name="s",
        num_cores=NC, num_subcores=NS)

    @functools.partial(
        pl.kernel,
        out_type=jax.ShapeDtypeStruct((out_rows, D), jnp.float32),
        mesh=mesh,
        scratch_types=[
            pltpu.VMEM_SHARED((n_tab, D), jnp.float32),
            pltpu.VMEM((per_w,), jnp.int32),
            pltpu.VMEM((per_w,), jnp.int32),
            pltpu.VMEM((n_chunks, CH), jnp.int32),
            pltpu.VMEM((NBR, CH, D), jnp.float32),
            pltpu.VMEM((NBR, CH, D), jnp.float32),
            pltpu.SemaphoreType.DMA((NBR,)),
            pltpu.SemaphoreType.DMA((NBR,)),
            pltpu.SemaphoreType.DMA((NBR,)),
        ],
    )
    def route_kernel(table_hbm, src_hbm, dst_hbm, pos_hbm, out_hbm,
                     tab_sh, src_v, dst_v, pos_v, rows_s, rows_d,
                     gsems, gsemd, ssem):
        wid = lax.axis_index("s") * NC + lax.axis_index("c")
        sid = lax.axis_index("s")
        base0 = wid * per_w
        pltpu.sync_copy(
            table_hbm.at[pl.ds(sid * tab_per_tile, tab_per_tile), :],
            tab_sh.at[pl.ds(sid * tab_per_tile, tab_per_tile), :])
        pltpu.sync_copy(src_hbm.at[pl.ds(base0, per_w)], src_v)
        pltpu.sync_copy(dst_hbm.at[pl.ds(base0, per_w)], dst_v)
        pltpu.sync_copy(pos_hbm.at[pl.ds(wid * n_chunks, n_chunks), :], pos_v)
        plsc.subcore_barrier()

        def issue_gathers(c, b):
            pltpu.async_copy(
                tab_sh.at[src_v.at[pl.ds(c * CH, CH)]],
                rows_s.at[b], gsems.at[b])
            pltpu.async_copy(
                tab_sh.at[dst_v.at[pl.ds(c * CH, CH)]],
                rows_d.at[b], gsemd.at[b])

        def wait_gathers(b):
            pltpu.make_async_copy(
                tab_sh.at[src_v.at[pl.ds(0, CH)]],
                rows_s.at[b], gsems.at[b]).wait()
            pltpu.make_async_copy(
                tab_sh.at[dst_v.at[pl.ds(0, CH)]],
                rows_d.at[b], gsemd.at[b]).wait()

        def add_rows(b):
            bs = rows_s.at[b]
            bd = rows_d.at[b]

            def row_body(r, carry):
                for k in range(D // 16):
                    bs[r, pl.ds(k * 16, 16)] = (
                        bs[r, pl.ds(k * 16, 16)] + bd[r, pl.ds(k * 16, 16)])
                return carry

            lax.fori_loop(0, CH, row_body, 0)

        def issue_scatter(c, b):
            pltpu.async_copy(
                rows_s.at[b], out_hbm.at[pos_v.at[c]], ssem.at[b])

        def wait_scatter(b):
            pltpu.make_async_copy(
                rows_s.at[b], out_hbm.at[pos_v.at[0]], ssem.at[b]).wait()

        for b in range(NBR):
            issue_gathers(b, b)

        def round_body(o, carry):
            c0 = o * NBR
            for b in range(NBR):
                wait_gathers(b)
                add_rows(b)
                issue_scatter(c0 + b, b)
            for b in range(NBR):
                wait_scatter(b)
                issue_gathers(c0 + NBR + b, b)
            return carry

        lax.fori_loop(0, n_rounds - 1, round_body, 0)

        c0 = (n_rounds - 1) * NBR
        for b in range(NBR):
            wait_gathers(b)
            add_rows(b)
            issue_scatter(c0 + b, b)
        for b in range(NBR):
            wait_scatter(b)

    return route_kernel(table, src_idx, dst_idx, pos2d)


def _sc_gather_rows(table, idx, rows_total, clamp_max=0):
    per_w = rows_total // NW
    n_chunks = per_w // CHUNK
    assert per_w % CHUNK == 0 and n_chunks % NBUF == 0
    n_rounds = n_chunks // NBUF
    mesh = plsc.VectorSubcoreMesh(
        core_axis_name="c", subcore_axis_name="s",
        num_cores=NC, num_subcores=NS)

    @functools.partial(
        pl.kernel,
        out_type=jax.ShapeDtypeStruct((rows_total, D), jnp.float32),
        mesh=mesh,
        scratch_types=[
            pltpu.VMEM((per_w,), jnp.int32),
            pltpu.VMEM((NBUF, CHUNK, D), jnp.float32),
            pltpu.SemaphoreType.DMA((NBUF,)),
            pltpu.SemaphoreType.DMA((NBUF,)),
        ],
    )
    def gather_kernel(table_hbm, idx_hbm, out_hbm, idx_v, rows_v, gsem, ssem):
        wid = lax.axis_index("s") * NC + lax.axis_index("c")
        base0 = wid * per_w
        pltpu.sync_copy(idx_hbm.at[pl.ds(base0, per_w)], idx_v)

        if clamp_max > 0:
            def clamp_body(i, carry):
                v = idx_v[pl.ds(i * 16, 16)]
                idx_v[pl.ds(i * 16, 16)] = jnp.minimum(
                    jnp.maximum(v, 0), clamp_max - 1)
                return carry

            lax.fori_loop(0, per_w // 16, clamp_body, 0)

        def issue_gather(c, b):
            pltpu.async_copy(
                table_hbm.at[idx_v.at[pl.ds(c * CHUNK, CHUNK)]],
                rows_v.at[b], gsem.at[b])

        def wait_gather(b):
            pltpu.make_async_copy(
                table_hbm.at[idx_v.at[pl.ds(0, CHUNK)]],
                rows_v.at[b], gsem.at[b]).wait()

        def issue_store(c, b):
            pltpu.async_copy(
                rows_v.at[b],
                out_hbm.at[pl.ds(base0 + c * CHUNK, CHUNK), :], ssem.at[b])

        def wait_store(b):
            pltpu.make_async_copy(
                rows_v.at[b],
                out_hbm.at[pl.ds(base0, CHUNK), :], ssem.at[b]).wait()

        for b in range(NBUF):
            issue_gather(b, b)

        def round_body(o, carry):
            c0 = o * NBUF
            for b in range(NBUF):
                wait_gather(b)
                issue_store(c0 + b, b)
            for b in range(NBUF):
                wait_store(b)
                issue_gather(c0 + NBUF + b, b)
            return carry

        lax.fori_loop(0, n_rounds - 1, round_body, 0)

        c0 = (n_rounds - 1) * NBUF
        for b in range(NBUF):
            wait_gather(b)
            issue_store(c0 + b, b)
        for b in range(NBUF):
            wait_store(b)

    return gather_kernel(table, idx)


def _mlp_body(be_ref, hs_ref, w1_ref, b1_ref, w2_ref, b2_ref, o_ref):
    h = hs_ref[...].astype(jnp.bfloat16)
    z = jnp.dot(h, w1_ref[0], preferred_element_type=jnp.float32)
    z = jnp.maximum(z + b1_ref[0], 0.0).astype(jnp.bfloat16)
    o_ref[...] = jnp.dot(z, w2_ref[0], preferred_element_type=jnp.float32) + b2_ref[0]


def _routed_mlp(block_expert, g, W1, b1, W2, b2):
    grid_spec = pltpu.PrefetchScalarGridSpec(
        num_scalar_prefetch=1,
        grid=(NB,),
        in_specs=[
            pl.BlockSpec((TE, D), lambda i, be: (i, 0)),
            pl.BlockSpec((1, D, H), lambda i, be: (be[i], 0, 0)),
            pl.BlockSpec((1, 1, H), lambda i, be: (be[i], 0, 0)),
            pl.BlockSpec((1, H, D), lambda i, be: (be[i], 0, 0)),
            pl.BlockSpec((1, 1, D), lambda i, be: (be[i], 0, 0)),
        ],
        out_specs=pl.BlockSpec((TE, D), lambda i, be: (i, 0)),
    )
    return pl.pallas_call(
        _mlp_body,
        grid_spec=grid_spec,
        out_shape=jax.ShapeDtypeStruct((EP, D), jnp.float32),
    )(block_expert, g, W1.astype(jnp.bfloat16), b1.reshape(NUM_E, 1, H),
      W2.astype(jnp.bfloat16), b2.reshape(NUM_E, 1, D))


def kernel(x, edge_index, edge_vec, W1, b1, W2, b2):
    src = edge_index[0]
    dst = edge_index[1]
    lengths = jnp.sqrt(jnp.sum(edge_vec * edge_vec, axis=-1))
    eid = (lengths >= 1.3).astype(jnp.int32) + (lengths >= 2.0).astype(jnp.int32)

    onehot = (eid[:, None] == jnp.arange(NUM_E, dtype=jnp.int32)[None, :])
    csum = jnp.cumsum(onehot.astype(jnp.int32), axis=0)
    counts = csum[-1]
    rank = jnp.take_along_axis(csum, eid[:, None], axis=1)[:, 0] - 1
    nb_g = (counts + TE - 1) // TE
    off = jnp.concatenate(
        [jnp.zeros((1,), jnp.int32), jnp.cumsum(nb_g[:2] * TE).astype(jnp.int32)])
    padded_pos = off[eid] + rank

    pad_n = NW * (-(-E // (NW * CH * 8)) * CH * 8) - E
    pos_cat = jnp.concatenate(
        [padded_pos, EP + jnp.arange(pad_n, dtype=jnp.int32)]).reshape(-1, CH)
    zpad = jnp.zeros((pad_n,), jnp.int32)
    x_pad = jnp.concatenate(
        [x, jnp.zeros((-N % (NS * 8), D), jnp.float32)])
    hsum = _sc_route_sum_rows(
        x_pad, jnp.concatenate([src, zpad]), jnp.concatenate([dst, zpad]),
        pos_cat, EP + pad_n)

    blk = jnp.arange(NB, dtype=jnp.int32) * TE
    block_expert = (blk >= off[1]).astype(jnp.int32) + (blk >= off[2]).astype(jnp.int32)

    out_padded = _routed_mlp(block_expert, hsum, W1, b1, W2, b2)

    gpos = jnp.concatenate([padded_pos, jnp.zeros((EP - E,), jnp.int32)])
    res_pad = _sc_gather_rows(out_padded, gpos, EP)
    return res_pad[:E]


# speedup vs baseline: 1.6917x; 1.1232x over previous
import functools

import jax
import jax.numpy as jnp
from jax import lax
from jax.experimental import pallas as pl
from jax.experimental.pallas import tpu as pltpu
from jax.experimental.pallas import tpu_sc as plsc

N = 10000
E = 160000
D = 128
H = 512
NUM_E = 3

TE = 1024
EP = 163840
NB = EP // TE

NC, NS = 2, 16
NW = NC * NS
CHUNK = 128


NBUF = 5


CH = 32


def _sc_route_sum_rows(table, src_idx, dst_idx, pos2d, out_rows):
    per_w = src_idx.shape[0] // NW
    n_chunks = per_w // CH
    assert per_w % CH == 0 and n_chunks % 8 == 0
    NBR = 2
    n_rounds = n_chunks // NBR
    n_tab = table.shape[0]
    assert n_tab % (NS * 8) == 0
    tab_per_tile = n_tab // NS
    mesh = plsc.VectorSubcoreMesh(
        core_axis_name="c", subcore_axis_name="s",
        num_cores=NC, num_subcores=NS)

    @functools.partial(
        pl.kernel,
        out_type=jax.ShapeDtypeStruct((out_rows, D), jnp.float32),
        mesh=mesh,
        scratch_types=[
            pltpu.VMEM_SHARED((n_tab, D), jnp.float32),
            pltpu.VMEM((per_w,), jnp.int32),
            pltpu.VMEM((per_w,), jnp.int32),
            pltpu.VMEM((n_chunks, CH), jnp.int32),
            pltpu.VMEM((NBR, CH, D), jnp.float32),
            pltpu.VMEM((NBR, CH, D), jnp.float32),
            pltpu.SemaphoreType.DMA((NBR,)),
            pltpu.SemaphoreType.DMA((NBR,)),
            pltpu.SemaphoreType.DMA((NBR,)),
        ],
    )
    def route_kernel(table_hbm, src_hbm, dst_hbm, pos_hbm, out_hbm,
                     tab_sh, src_v, dst_v, pos_v, rows_s, rows_d,
                     gsems, gsemd, ssem):
        wid = lax.axis_index("s") * NC + lax.axis_index("c")
        sid = lax.axis_index("s")
        base0 = wid * per_w
        pltpu.sync_copy(
            table_hbm.at[pl.ds(sid * tab_per_tile, tab_per_tile), :],
            tab_sh.at[pl.ds(sid * tab_per_tile, tab_per_tile), :])
        pltpu.sync_copy(src_hbm.at[pl.ds(base0, per_w)], src_v)
        pltpu.sync_copy(dst_hbm.at[pl.ds(base0, per_w)], dst_v)
        pltpu.sync_copy(pos_hbm.at[pl.ds(wid * n_chunks, n_chunks), :], pos_v)
        plsc.subcore_barrier()

        def issue_gathers(c, b):
            pltpu.async_copy(
                tab_sh.at[src_v.at[pl.ds(c * CH, CH)]],
                rows_s.at[b], gsems.at[b])
            pltpu.async_copy(
                tab_sh.at[dst_v.at[pl.ds(c * CH, CH)]],
                rows_d.at[b], gsemd.at[b])

        def wait_gathers(b):
            pltpu.make_async_copy(
                tab_sh.at[src_v.at[pl.ds(0, CH)]],
                rows_s.at[b], gsems.at[b]).wait()
            pltpu.make_async_copy(
                tab_sh.at[dst_v.at[pl.ds(0, CH)]],
                rows_d.at[b], gsemd.at[b]).wait()

        def add_rows(b):
            bs = rows_s.at[b]
            bd = rows_d.at[b]

            def row_body(r, carry):
                for k in range(D // 16):
                    bs[r, pl.ds(k * 16, 16)] = (
                        bs[r, pl.ds(k * 16, 16)] + bd[r, pl.ds(k * 16, 16)])
                return carry

            lax.fori_loop(0, CH, row_body, 0)

        def issue_scatter(c, b):
            pltpu.async_copy(
                rows_s.at[b], out_hbm.at[pos_v.at[c]], ssem.at[b])

        def wait_scatter(b):
            pltpu.make_async_copy(
                rows_s.at[b], out_hbm.at[pos_v.at[0]], ssem.at[b]).wait()

        for b in range(NBR):
            issue_gathers(b, b)

        def round_body(o, carry):
            c0 = o * NBR
            for b in range(NBR):
                wait_gathers(b)
                add_rows(b)
                issue_scatter(c0 + b, b)
            for b in range(NBR):
                wait_scatter(b)
                issue_gathers(c0 + NBR + b, b)
            return carry

        lax.fori_loop(0, n_rounds - 1, round_body, 0)

        c0 = (n_rounds - 1) * NBR
        for b in range(NBR):
            wait_gathers(b)
            add_rows(b)
            issue_scatter(c0 + b, b)
        for b in range(NBR):
            wait_scatter(b)

    return route_kernel(table, src_idx, dst_idx, pos2d)


def _sc_gather_rows(table, idx, rows_total, clamp_max=0):
    per_w = rows_total // NW
    n_chunks = per_w // CHUNK
    assert per_w % CHUNK == 0 and n_chunks % NBUF == 0
    n_rounds = n_chunks // NBUF
    mesh = plsc.VectorSubcoreMesh(
        core_axis_name="c", subcore_axis_name="s",
        num_cores=NC, num_subcores=NS)

    @functools.partial(
        pl.kernel,
        out_type=jax.ShapeDtypeStruct((rows_total, D), jnp.float32),
        mesh=mesh,
        scratch_types=[
            pltpu.VMEM((per_w,), jnp.int32),
            pltpu.VMEM((NBUF, CHUNK, D), jnp.float32),
            pltpu.SemaphoreType.DMA((NBUF,)),
            pltpu.SemaphoreType.DMA((NBUF,)),
        ],
    )
    def gather_kernel(table_hbm, idx_hbm, out_hbm, idx_v, rows_v, gsem, ssem):
        wid = lax.axis_index("s") * NC + lax.axis_index("c")
        base0 = wid * per_w
        pltpu.sync_copy(idx_hbm.at[pl.ds(base0, per_w)], idx_v)

        if clamp_max > 0:
            def clamp_body(i, carry):
                v = idx_v[pl.ds(i * 16, 16)]
                idx_v[pl.ds(i * 16, 16)] = jnp.minimum(
                    jnp.maximum(v, 0), clamp_max - 1)
                return carry

            lax.fori_loop(0, per_w // 16, clamp_body, 0)

        def issue_gather(c, b):
            pltpu.async_copy(
                table_hbm.at[idx_v.at[pl.ds(c * CHUNK, CHUNK)]],
                rows_v.at[b], gsem.at[b])

        def wait_gather(b):
            pltpu.make_async_copy(
                table_hbm.at[idx_v.at[pl.ds(0, CHUNK)]],
                rows_v.at[b], gsem.at[b]).wait()

        def issue_store(c, b):
            pltpu.async_copy(
                rows_v.at[b],
                out_hbm.at[pl.ds(base0 + c * CHUNK, CHUNK), :], ssem.at[b])

        def wait_store(b):
            pltpu.make_async_copy(
                rows_v.at[b],
                out_hbm.at[pl.ds(base0, CHUNK), :], ssem.at[b]).wait()

        for b in range(NBUF):
            issue_gather(b, b)

        def round_body(o, carry):
            c0 = o * NBUF
            for b in range(NBUF):
                wait_gather(b)
                issue_store(c0 + b, b)
            for b in range(NBUF):
                wait_store(b)
                issue_gather(c0 + NBUF + b, b)
            return carry

        lax.fori_loop(0, n_rounds - 1, round_body, 0)

        c0 = (n_rounds - 1) * NBUF
        for b in range(NBUF):
            wait_gather(b)
            issue_store(c0 + b, b)
        for b in range(NBUF):
            wait_store(b)

    return gather_kernel(table, idx)


def _mlp_body(be_ref, hs_ref, w1_ref, b1_ref, w2_ref, b2_ref, o_ref):
    h = hs_ref[...].astype(jnp.bfloat16)
    z = jnp.dot(h, w1_ref[0], preferred_element_type=jnp.float32)
    z = jnp.maximum(z + b1_ref[0], 0.0).astype(jnp.bfloat16)
    o_ref[...] = jnp.dot(z, w2_ref[0], preferred_element_type=jnp.float32) + b2_ref[0]


def _routed_mlp(block_expert, g, W1, b1, W2, b2):
    grid_spec = pltpu.PrefetchScalarGridSpec(
        num_scalar_prefetch=1,
        grid=(NB,),
        in_specs=[
            pl.BlockSpec((TE, D), lambda i, be: (i, 0)),
            pl.BlockSpec((1, D, H), lambda i, be: (be[i], 0, 0)),
            pl.BlockSpec((1, 1, H), lambda i, be: (be[i], 0, 0)),
            pl.BlockSpec((1, H, D), lambda i, be: (be[i], 0, 0)),
            pl.BlockSpec((1, 1, D), lambda i, be: (be[i], 0, 0)),
        ],
        out_specs=pl.BlockSpec((TE, D), lambda i, be: (i, 0)),
    )
    return pl.pallas_call(
        _mlp_body,
        grid_spec=grid_spec,
        out_shape=jax.ShapeDtypeStruct((EP, D), jnp.float32),
    )(block_expert, g, W1.astype(jnp.bfloat16), b1.reshape(NUM_E, 1, H),
      W2.astype(jnp.bfloat16), b2.reshape(NUM_E, 1, D))


def kernel(x, edge_index, edge_vec, W1, b1, W2, b2):
    src = edge_index[0]
    dst = edge_index[1]
    lengths = jnp.sqrt(jnp.sum(edge_vec * edge_vec, axis=-1))
    eid = (lengths >= 1.3).astype(jnp.int32) + (lengths >= 2.0).astype(jnp.int32)

    onehot = (eid[:, None] == jnp.arange(NUM_E, dtype=jnp.int32)[None, :])
    csum = jnp.cumsum(onehot.astype(jnp.int32), axis=0)
    counts = csum[-1]
    rank = jnp.take_along_axis(csum, eid[:, None], axis=1)[:, 0] - 1
    nb_g = (counts + TE - 1) // TE
    off = jnp.concatenate(
        [jnp.zeros((1,), jnp.int32), jnp.cumsum(nb_g[:2] * TE).astype(jnp.int32)])
    padded_pos = off[eid] + rank

    pad_n = NW * (-(-E // (NW * CH * 8)) * CH * 8) - E
    pos_cat = jnp.concatenate(
        [padded_pos, EP + jnp.arange(pad_n, dtype=jnp.int32)]).reshape(-1, CH)
    zpad = jnp.zeros((pad_n,), jnp.int32)
    x_pad = jnp.concatenate(
        [x, jnp.zeros((-N % (NS * 8), D), jnp.float32)])
    hsum = _sc_route_sum_rows(
        x_pad, jnp.concatenate([src, zpad]), jnp.concatenate([dst, zpad]),
        pos_cat, EP + pad_n)

    blk = jnp.arange(NB, dtype=jnp.int32) * TE
    block_expert = (blk >= off[1]).astype(jnp.int32) + (blk >= off[2]).astype(jnp.int32)

    out_padded = _routed_mlp(block_expert, hsum, W1, b1, W2, b2)

    gpos = jnp.concatenate([padded_pos, jnp.zeros((EP - E,), jnp.int32)])
    res_pad = _sc_gather_rows(out_padded, gpos, EP)
    return res_pad[:E]
